# baseline (speedup 1.0000x reference)
import functools

import jax
import jax.numpy as jnp
from jax import lax
from jax.experimental import pallas as pl
from jax.experimental.pallas import tpu as pltpu
from jax.experimental.pallas import tpu_sc as plsc

N_NODES = 10000
N_EDGES = 320000
D = 128
NC = 2
NS = 16
CHUNK = 128
CPT = 160
EPAD = NS * CPT * CHUNK
HALF = 5056
ACCR = 5504
STRIPE = ACCR // NS
NDUM = 448
CW = 16

_MESH = plsc.VectorSubcoreMesh(core_axis_name="c", subcore_axis_name="s")


def _sc_cnt_fn(dstg, zacc, ones_hbm, dep,
               ocnt,
               didx, ones_v, cnt_sh, sem):
    del dep
    c = lax.axis_index("c")
    s = lax.axis_index("s")
    r0 = s * STRIPE
    pltpu.sync_copy(zacc.at[pl.ds(r0, STRIPE)], cnt_sh.at[pl.ds(r0, STRIPE)])
    pltpu.sync_copy(ones_hbm, ones_v)
    pltpu.sync_copy(dstg.at[c, s], didx)
    plsc.subcore_barrier()

    def body(j, carry):
        pltpu.sync_copy(ones_v, cnt_sh.at[didx.at[j]], add=True)
        return carry

    lax.fori_loop(0, CPT, body, 0)
    plsc.subcore_barrier()
    pltpu.sync_copy(cnt_sh.at[pl.ds(r0, STRIPE)], ocnt.at[c, pl.ds(r0, STRIPE)])


_sc_cnt = pl.kernel(
    _sc_cnt_fn,
    out_type=[jax.ShapeDtypeStruct((NC, ACCR, D), jnp.float32)],
    mesh=_MESH,
    scratch_types=[
        pltpu.VMEM((CPT, CHUNK), jnp.int32),
        pltpu.VMEM((CHUNK, D), jnp.float32),
        pltpu.VMEM_SHARED((ACCR, D), jnp.float32),
        pltpu.SemaphoreType.DMA,
    ],
)


NBUF = 2


def _sc_agg_fn(x_hbm, srcg, dstg, zacc,
               osum,
               sidx, didx, rows0, rows1, acc_sh,
               gsem0, gsem1, ssem0, ssem1):
    c = lax.axis_index("c")
    s = lax.axis_index("s")
    r0 = s * STRIPE
    pltpu.sync_copy(zacc.at[pl.ds(r0, STRIPE)], acc_sh.at[pl.ds(r0, STRIPE)])
    pltpu.sync_copy(srcg.at[s], sidx)
    pltpu.sync_copy(dstg.at[c, s], didx)
    plsc.subcore_barrier()

    rows = [rows0, rows1]
    gsem = [gsem0, gsem1]
    ssem = [ssem0, ssem1]
    NSPL = 4
    SUB = CHUNK // NSPL

    def start_g(j, b):
        return [
            pltpu.async_copy(
                x_hbm.at[sidx.at[j, pl.ds(k * SUB, SUB)]],
                rows[b].at[pl.ds(k * SUB, SUB)],
                gsem[b],
            )
            for k in range(NSPL)
        ]

    gh = [None] * NBUF
    sh = [None] * NBUF
    gh[0] = start_g(0, 0)
    for j in range(CPT):
        b = j % NBUF
        if j + 1 < CPT:
            nb = (j + 1) % NBUF
            if sh[nb] is not None:
                sh[nb].wait()
                sh[nb] = None
            gh[nb] = start_g(j + 1, nb)
        for h in gh[b]:
            h.wait()
        sh[b] = pltpu.async_copy(rows[b], acc_sh.at[didx.at[j]], ssem[b], add=True)
    for b in range(NBUF):
        if sh[b] is not None:
            sh[b].wait()
    plsc.subcore_barrier()
    pltpu.sync_copy(acc_sh.at[pl.ds(r0, STRIPE)], osum.at[c, pl.ds(r0, STRIPE)])


_sc_agg = pl.kernel(
    _sc_agg_fn,
    out_type=[jax.ShapeDtypeStruct((NC, ACCR, D), jnp.float32)],
    mesh=_MESH,
    scratch_types=[
        pltpu.VMEM((CPT, CHUNK), jnp.int32),
        pltpu.VMEM((CPT, CHUNK), jnp.int32),
        pltpu.VMEM((CHUNK, D), jnp.float32),
        pltpu.VMEM((CHUNK, D), jnp.float32),
        pltpu.VMEM_SHARED((ACCR, D), jnp.float32),
        pltpu.SemaphoreType.DMA,
        pltpu.SemaphoreType.DMA,
        pltpu.SemaphoreType.DMA,
        pltpu.SemaphoreType.DMA,
    ],
)


def _dense_fn(relu, p, cc, xin, wl, wr, b, out):
    cnt = jnp.maximum(cc[...], 1.0)
    mean = p[...] / cnt
    acc = (jnp.dot(mean, wl[...], preferred_element_type=jnp.float32)
           + jnp.dot(xin[...], wr[...], preferred_element_type=jnp.float32)
           + b[...])
    out[...] = jnp.maximum(acc, 0.0) if relu else acc


_ROWS = 2000


def _dense(p, cc, xin, wlT, wrT, b, relu):
    return pl.pallas_call(
        functools.partial(_dense_fn, relu),
        grid=(N_NODES // _ROWS,),
        in_specs=[
            pl.BlockSpec((_ROWS, D), lambda i: (i, 0)),
            pl.BlockSpec((_ROWS, 1), lambda i: (i, 0)),
            pl.BlockSpec((_ROWS, D), lambda i: (i, 0)),
            pl.BlockSpec((D, D), lambda i: (0, 0)),
            pl.BlockSpec((D, D), lambda i: (0, 0)),
            pl.BlockSpec((1, D), lambda i: (0, 0)),
        ],
        out_specs=pl.BlockSpec((_ROWS, D), lambda i: (i, 0)),
        out_shape=jax.ShapeDtypeStruct((N_NODES, D), jnp.float32),
    )(p, cc, xin, wlT, wrT, b)


def kernel(x, edge_index, W1_l, b1_l, W1_r, W2_l, b2_l, W2_r):
    ei = edge_index.astype(jnp.int32)
    pad = EPAD - N_EDGES
    src = jnp.concatenate([ei[0], jnp.zeros((pad,), jnp.int32)])
    dst = jnp.concatenate([ei[1], jnp.full((pad,), N_NODES, jnp.int32)])
    dummy = HALF + (jnp.arange(EPAD, dtype=jnp.int32) % NDUM)
    srcg = src.reshape(NS, CPT, CHUNK)
    dloc = [jnp.where((dst >= c * HALF) & (dst < (c + 1) * HALF),
                      dst - c * HALF, dummy).reshape(NS, CPT, CHUNK)
            for c in range(NC)]
    dstg = jnp.stack(dloc)
    zacc = jnp.zeros((ACCR, D), jnp.float32)

    ones = jnp.ones((CHUNK, D), jnp.float32)
    (osum,) = _sc_agg(x, srcg, dstg, zacc)
    sums = jnp.concatenate([osum[0, :HALF], osum[1, :HALF]])[:N_NODES]
    (ocnt,) = _sc_cnt(dstg, zacc, ones, osum)
    cnts = jnp.concatenate([ocnt[0, :HALF], ocnt[1, :HALF]])[:N_NODES, 0:1]
    h = _dense(sums, cnts, x, W1_l.T, W1_r.T, b1_l[None, :], relu=True)
    (osum2,) = _sc_agg(h, srcg, dstg, zacc)
    sums2 = jnp.concatenate([osum2[0, :HALF], osum2[1, :HALF]])[:N_NODES]
    out = _dense(sums2, cnts, h, W2_l.T, W2_r.T, b2_l[None, :], relu=False)
    return out


# speedup vs baseline: 1.0051x; 1.0051x over previous
import functools

import jax
import jax.numpy as jnp
from jax import lax
from jax.experimental import pallas as pl
from jax.experimental.pallas import tpu as pltpu
from jax.experimental.pallas import tpu_sc as plsc

N_NODES = 10000
N_EDGES = 320000
D = 128
NC = 2
NS = 16
CHUNK = 128
CPT = 160
EPAD = NS * CPT * CHUNK
HALF = 5056
ACCR = 5504
STRIPE = ACCR // NS
NDUM = 448
CW = 16

_MESH = plsc.VectorSubcoreMesh(core_axis_name="c", subcore_axis_name="s")


def _sc_cnt_fn(dstg, zacc, ones_hbm, dep,
               ocnt,
               didx, ones_v, cnt_sh, sem):
    del dep
    c = lax.axis_index("c")
    s = lax.axis_index("s")
    r0 = s * STRIPE
    pltpu.sync_copy(zacc.at[pl.ds(r0, STRIPE)], cnt_sh.at[pl.ds(r0, STRIPE)])
    pltpu.sync_copy(ones_hbm, ones_v)
    pltpu.sync_copy(dstg.at[c, s], didx)
    plsc.subcore_barrier()

    def body(j, carry):
        pltpu.sync_copy(ones_v, cnt_sh.at[didx.at[j]], add=True)
        return carry

    lax.fori_loop(0, CPT, body, 0)
    plsc.subcore_barrier()
    pltpu.sync_copy(cnt_sh.at[pl.ds(r0, STRIPE)], ocnt.at[c, pl.ds(r0, STRIPE)])


_sc_cnt = pl.kernel(
    _sc_cnt_fn,
    out_type=[jax.ShapeDtypeStruct((NC, ACCR, D), jnp.float32)],
    mesh=_MESH,
    scratch_types=[
        pltpu.VMEM((CPT, CHUNK), jnp.int32),
        pltpu.VMEM((CHUNK, D), jnp.float32),
        pltpu.VMEM_SHARED((ACCR, D), jnp.float32),
        pltpu.SemaphoreType.DMA,
    ],
)


NBUF = 2


def _sc_agg_fn(x_hbm, srcg, dstg, zacc,
               osum,
               sidx, didx, rows0, rows1, acc_sh,
               gsem0, gsem1, ssem0, ssem1):
    c = lax.axis_index("c")
    s = lax.axis_index("s")
    r0 = s * STRIPE
    pltpu.sync_copy(zacc.at[pl.ds(r0, STRIPE)], acc_sh.at[pl.ds(r0, STRIPE)])
    pltpu.sync_copy(srcg.at[s], sidx)
    pltpu.sync_copy(dstg.at[c, s], didx)
    plsc.subcore_barrier()

    rows = [rows0, rows1]
    gsem = [gsem0, gsem1]
    ssem = [ssem0, ssem1]
    gh = [None] * NBUF
    sh = [None] * NBUF
    gh[0] = pltpu.async_copy(x_hbm.at[sidx.at[0]], rows[0], gsem[0])
    for j in range(CPT):
        b = j % NBUF
        if j + 1 < CPT:
            nb = (j + 1) % NBUF
            if sh[nb] is not None:
                sh[nb].wait()
                sh[nb] = None
            gh[nb] = pltpu.async_copy(x_hbm.at[sidx.at[j + 1]], rows[nb], gsem[nb])
        gh[b].wait()
        sh[b] = pltpu.async_copy(rows[b], acc_sh.at[didx.at[j]], ssem[b], add=True)
    for b in range(NBUF):
        if sh[b] is not None:
            sh[b].wait()
    plsc.subcore_barrier()
    pltpu.sync_copy(acc_sh.at[pl.ds(r0, STRIPE)], osum.at[c, pl.ds(r0, STRIPE)])


_sc_agg = pl.kernel(
    _sc_agg_fn,
    out_type=[jax.ShapeDtypeStruct((NC, ACCR, D), jnp.float32)],
    mesh=_MESH,
    scratch_types=[
        pltpu.VMEM((CPT, CHUNK), jnp.int32),
        pltpu.VMEM((CPT, CHUNK), jnp.int32),
        pltpu.VMEM((CHUNK, D), jnp.float32),
        pltpu.VMEM((CHUNK, D), jnp.float32),
        pltpu.VMEM_SHARED((ACCR, D), jnp.float32),
        pltpu.SemaphoreType.DMA,
        pltpu.SemaphoreType.DMA,
        pltpu.SemaphoreType.DMA,
        pltpu.SemaphoreType.DMA,
    ],
)


def _dense_fn(relu, p, cc, xin, wl, wr, b, out):
    cnt = jnp.maximum(cc[...], 1.0)
    mean = p[...] / cnt
    acc = (jnp.dot(mean, wl[...], preferred_element_type=jnp.float32)
           + jnp.dot(xin[...], wr[...], preferred_element_type=jnp.float32)
           + b[...])
    out[...] = jnp.maximum(acc, 0.0) if relu else acc


_ROWS = 2000


def _dense(p, cc, xin, wlT, wrT, b, relu):
    return pl.pallas_call(
        functools.partial(_dense_fn, relu),
        grid=(N_NODES // _ROWS,),
        in_specs=[
            pl.BlockSpec((_ROWS, D), lambda i: (i, 0)),
            pl.BlockSpec((_ROWS, 1), lambda i: (i, 0)),
            pl.BlockSpec((_ROWS, D), lambda i: (i, 0)),
            pl.BlockSpec((D, D), lambda i: (0, 0)),
            pl.BlockSpec((D, D), lambda i: (0, 0)),
            pl.BlockSpec((1, D), lambda i: (0, 0)),
        ],
        out_specs=pl.BlockSpec((_ROWS, D), lambda i: (i, 0)),
        out_shape=jax.ShapeDtypeStruct((N_NODES, D), jnp.float32),
    )(p, cc, xin, wlT, wrT, b)


def kernel(x, edge_index, W1_l, b1_l, W1_r, W2_l, b2_l, W2_r):
    ei = edge_index.astype(jnp.int32)
    pad = EPAD - N_EDGES
    src = jnp.concatenate([ei[0], jnp.zeros((pad,), jnp.int32)])
    dst = jnp.concatenate([ei[1], jnp.full((pad,), N_NODES, jnp.int32)])
    dummy = HALF + (jnp.arange(EPAD, dtype=jnp.int32) % NDUM)
    srcg = src.reshape(NS, CPT, CHUNK)
    dloc = [jnp.where((dst >= c * HALF) & (dst < (c + 1) * HALF),
                      dst - c * HALF, dummy).reshape(NS, CPT, CHUNK)
            for c in range(NC)]
    dstg = jnp.stack(dloc)
    zacc = jnp.zeros((ACCR, D), jnp.float32)

    ones = jnp.ones((CHUNK, D), jnp.float32)
    (osum,) = _sc_agg(x, srcg, dstg, zacc)
    sums = jnp.concatenate([osum[0, :HALF], osum[1, :HALF]])[:N_NODES]
    (ocnt,) = _sc_cnt(dstg, zacc, ones, osum)
    cnts = jnp.concatenate([ocnt[0, :HALF], ocnt[1, :HALF]])[:N_NODES, 0:1]
    h = _dense(sums, cnts, x, W1_l.T, W1_r.T, b1_l[None, :], relu=True)
    (osum2,) = _sc_agg(h, srcg, dstg, zacc)
    sums2 = jnp.concatenate([osum2[0, :HALF], osum2[1, :HALF]])[:N_NODES]
    out = _dense(sums2, cnts, h, W2_l.T, W2_r.T, b2_l[None, :], relu=False)
    return out
